# Initial kernel scaffold; baseline (speedup 1.0000x reference)
import functools

import jax
import jax.numpy as jnp
from jax.experimental import pallas as pl
from jax.experimental.pallas import tpu as pltpu

S2 = 2048
D = 1024
E = 64
TOPK = 8
NG = 8
TG = 4
DFF = 512
RSF = 2.5
SDFF = 1024

_NEG = jnp.float32(-1e30)


def _router_body(x_ref, gw_ref, b_ref, w_ref):
    x = x_ref[...]
    gw = gw_ref[...]
    logits = jax.lax.dot_general(
        x, gw, (((1,), (1,)), ((), ())),
        preferred_element_type=jnp.float32,
        precision=jax.lax.Precision.HIGHEST,
    )
    scores = jax.nn.sigmoid(logits)
    sc = scores + b_ref[...]
    cols = jax.lax.broadcasted_iota(jnp.int32, (S2, E), 1)
    grp = cols // (E // NG)

    gs_full = jnp.zeros_like(sc)
    for g in range(NG):
        ing = grp == g
        vals = jnp.where(ing, sc, _NEG)
        m1 = jnp.max(vals, axis=-1, keepdims=True)
        i1 = jnp.min(jnp.where(vals == m1, cols, 9999), axis=-1, keepdims=True)
        m2 = jnp.max(jnp.where(cols == i1, _NEG, vals), axis=-1, keepdims=True)
        gs_full = gs_full + jnp.where(ing, m1 + m2, 0.0)

    gsr = gs_full
    chosen = jnp.zeros_like(sc, dtype=jnp.bool_)
    for _ in range(TG):
        m = jnp.max(gsr, axis=-1, keepdims=True)
        gidx = jnp.min(jnp.where(gsr == m, grp, 9999), axis=-1, keepdims=True)
        ch = grp == gidx
        chosen = jnp.logical_or(chosen, ch)
        gsr = jnp.where(ch, _NEG, gsr)

    tmp = jnp.where(chosen, sc, 0.0)
    sel = jnp.zeros_like(sc, dtype=jnp.bool_)
    for _ in range(TOPK):
        cur = jnp.where(sel, _NEG, tmp)
        m = jnp.max(cur, axis=-1, keepdims=True)
        ik = jnp.min(jnp.where(cur == m, cols, 9999), axis=-1, keepdims=True)
        sel = jnp.logical_or(sel, cols == ik)

    wraw = jnp.where(sel, scores, 0.0)
    wsum = jnp.sum(wraw, axis=-1, keepdims=True)
    w_ref[...] = wraw * (RSF / (wsum + 1e-20))


def _shared_body(x_ref, sg_ref, su_ref, sd_ref, out_ref):
    xb = x_ref[...]
    a = jax.lax.dot_general(xb, sg_ref[...].astype(jnp.bfloat16),
                            (((1,), (1,)), ((), ())),
                            preferred_element_type=jnp.float32)
    b = jax.lax.dot_general(xb, su_ref[...].astype(jnp.bfloat16),
                            (((1,), (1,)), ((), ())),
                            preferred_element_type=jnp.float32)
    h = (a * jax.nn.sigmoid(a)) * b
    out_ref[...] = jax.lax.dot_general(h.astype(jnp.bfloat16),
                                       sd_ref[...].astype(jnp.bfloat16),
                                       (((1,), (1,)), ((), ())),
                                       preferred_element_type=jnp.float32)


def _sweep_body(w_ref, x_ref, g_ref, u_ref, d_ref, shared_ref, out_ref, acc_ref):
    e = pl.program_id(0)

    @pl.when(e == 0)
    def _():
        acc_ref[...] = shared_ref[...]

    xb = x_ref[...]
    g = g_ref[0].astype(jnp.bfloat16)
    u = u_ref[0].astype(jnp.bfloat16)
    a = jax.lax.dot_general(xb, g, (((1,), (1,)), ((), ())),
                            preferred_element_type=jnp.float32)
    b = jax.lax.dot_general(xb, u, (((1,), (1,)), ((), ())),
                            preferred_element_type=jnp.float32)
    h = (a * jax.nn.sigmoid(a)) * b
    d = d_ref[0].astype(jnp.bfloat16)
    o = jax.lax.dot_general(h.astype(jnp.bfloat16), d,
                            (((1,), (1,)), ((), ())),
                            preferred_element_type=jnp.float32)
    cols = jax.lax.broadcasted_iota(jnp.int32, (S2, E), 1)
    w_col = jnp.sum(jnp.where(cols == e, w_ref[...], 0.0), axis=-1,
                    keepdims=True)
    acc_ref[...] = acc_ref[...] + o * w_col

    @pl.when(e == E - 1)
    def _():
        out_ref[...] = acc_ref[...]


def kernel(hidden_states, gate_weight, e_score_correction_bias, gate_proj,
           up_proj, down_proj, shared_gate, shared_up, shared_down):
    x = hidden_states.reshape(S2, D).astype(jnp.float32)
    bias2d = e_score_correction_bias.reshape(1, E)

    w_dense = pl.pallas_call(
        _router_body,
        out_shape=jax.ShapeDtypeStruct((S2, E), jnp.float32),
    )(x, gate_weight, bias2d)

    xb = x.astype(jnp.bfloat16)

    shared_out = pl.pallas_call(
        _shared_body,
        grid=(8,),
        in_specs=[
            pl.BlockSpec((S2 // 8, D), lambda i: (i, 0)),
            pl.BlockSpec((SDFF, D), lambda i: (0, 0)),
            pl.BlockSpec((SDFF, D), lambda i: (0, 0)),
            pl.BlockSpec((D, SDFF), lambda i: (0, 0)),
        ],
        out_specs=pl.BlockSpec((S2 // 8, D), lambda i: (i, 0)),
        out_shape=jax.ShapeDtypeStruct((S2, D), jnp.float32),
    )(xb, shared_gate, shared_up, shared_down)

    out = pl.pallas_call(
        _sweep_body,
        grid=(E,),
        in_specs=[
            pl.BlockSpec((S2, E), lambda e: (0, 0)),
            pl.BlockSpec((S2, D), lambda e: (0, 0)),
            pl.BlockSpec((1, DFF, D), lambda e: (e, 0, 0)),
            pl.BlockSpec((1, DFF, D), lambda e: (e, 0, 0)),
            pl.BlockSpec((1, D, DFF), lambda e: (e, 0, 0)),
            pl.BlockSpec((S2, D), lambda e: (0, 0)),
        ],
        out_specs=pl.BlockSpec((S2, D), lambda e: (0, 0)),
        out_shape=jax.ShapeDtypeStruct((S2, D), jnp.float32),
        scratch_shapes=[pltpu.VMEM((S2, D), jnp.float32)],
    )(w_dense, xb, gate_proj, up_proj, down_proj, shared_out)

    return out.reshape(1, S2, D)


# speedup vs baseline: 21.2423x; 21.2423x over previous
import functools

import jax
import jax.numpy as jnp
from jax.experimental import pallas as pl
from jax.experimental.pallas import tpu as pltpu

S2 = 2048
D = 1024
E = 64
TOPK = 8
NG = 8
TG = 4
DFF = 512
RSF = 2.5
SDFF = 1024

_NEG = -1e30


def _router_body(x_ref, gw_ref, b_ref, w_ref):
    x = x_ref[...]
    gw = gw_ref[...]
    logits = jax.lax.dot_general(
        x.astype(jnp.bfloat16), gw.astype(jnp.bfloat16),
        (((1,), (1,)), ((), ())),
        preferred_element_type=jnp.float32,
    )
    scores = jax.nn.sigmoid(logits)
    sc = scores + b_ref[...]
    cols = jax.lax.broadcasted_iota(jnp.int32, (S2, E), 1)
    grp = cols // (E // NG)

    gs_full = jnp.zeros_like(sc)
    for g in range(NG):
        ing = grp == g
        vals = jnp.where(ing, sc, _NEG)
        m1 = jnp.max(vals, axis=-1, keepdims=True)
        i1 = jnp.min(jnp.where(vals == m1, cols, 9999), axis=-1, keepdims=True)
        m2 = jnp.max(jnp.where(cols == i1, _NEG, vals), axis=-1, keepdims=True)
        gs_full = gs_full + jnp.where(ing, m1 + m2, 0.0)

    gsr = gs_full
    chosen = jnp.zeros_like(sc, dtype=jnp.bool_)
    for _ in range(TG):
        m = jnp.max(gsr, axis=-1, keepdims=True)
        gidx = jnp.min(jnp.where(gsr == m, grp, 9999), axis=-1, keepdims=True)
        ch = grp == gidx
        chosen = jnp.logical_or(chosen, ch)
        gsr = jnp.where(ch, _NEG, gsr)

    tmp = jnp.where(chosen, sc, 0.0)
    sel = jnp.zeros_like(sc, dtype=jnp.bool_)
    for _ in range(TOPK):
        cur = jnp.where(sel, _NEG, tmp)
        m = jnp.max(cur, axis=-1, keepdims=True)
        ik = jnp.min(jnp.where(cur == m, cols, 9999), axis=-1, keepdims=True)
        sel = jnp.logical_or(sel, cols == ik)

    wraw = jnp.where(sel, scores, 0.0)
    wsum = jnp.sum(wraw, axis=-1, keepdims=True)
    w_ref[...] = wraw * (RSF / (wsum + 1e-20))


def _shared_body(x_ref, sg_ref, su_ref, sd_ref, out_ref):
    xb = x_ref[...]
    a = jax.lax.dot_general(xb, sg_ref[...].astype(jnp.bfloat16),
                            (((1,), (1,)), ((), ())),
                            preferred_element_type=jnp.float32)
    b = jax.lax.dot_general(xb, su_ref[...].astype(jnp.bfloat16),
                            (((1,), (1,)), ((), ())),
                            preferred_element_type=jnp.float32)
    h = (a * jax.nn.sigmoid(a)) * b
    out_ref[...] = jax.lax.dot_general(h.astype(jnp.bfloat16),
                                       sd_ref[...].astype(jnp.bfloat16),
                                       (((1,), (1,)), ((), ())),
                                       preferred_element_type=jnp.float32)


def _sweep_body(w_ref, x_ref, g_ref, u_ref, d_ref, shared_ref, out_ref, acc_ref):
    e = pl.program_id(0)

    @pl.when(e == 0)
    def _():
        acc_ref[...] = shared_ref[...]

    xb = x_ref[...]
    g = g_ref[0].astype(jnp.bfloat16)
    u = u_ref[0].astype(jnp.bfloat16)
    a = jax.lax.dot_general(xb, g, (((1,), (1,)), ((), ())),
                            preferred_element_type=jnp.float32)
    b = jax.lax.dot_general(xb, u, (((1,), (1,)), ((), ())),
                            preferred_element_type=jnp.float32)
    h = (a * jax.nn.sigmoid(a)) * b
    d = d_ref[0].astype(jnp.bfloat16)
    o = jax.lax.dot_general(h.astype(jnp.bfloat16), d,
                            (((1,), (1,)), ((), ())),
                            preferred_element_type=jnp.float32)
    cols = jax.lax.broadcasted_iota(jnp.int32, (S2, E), 1)
    w_col = jnp.sum(jnp.where(cols == e, w_ref[...], 0.0), axis=-1,
                    keepdims=True)
    acc_ref[...] = acc_ref[...] + o * w_col

    @pl.when(e == E - 1)
    def _():
        out_ref[...] = acc_ref[...]


def kernel(hidden_states, gate_weight, e_score_correction_bias, gate_proj,
           up_proj, down_proj, shared_gate, shared_up, shared_down):
    x = hidden_states.reshape(S2, D).astype(jnp.float32)
    bias2d = e_score_correction_bias.reshape(1, E)

    w_dense = pl.pallas_call(
        _router_body,
        out_shape=jax.ShapeDtypeStruct((S2, E), jnp.float32),
    )(x, gate_weight, bias2d)

    xb = x.astype(jnp.bfloat16)

    shared_out = pl.pallas_call(
        _shared_body,
        grid=(8,),
        in_specs=[
            pl.BlockSpec((S2 // 8, D), lambda i: (i, 0)),
            pl.BlockSpec((SDFF, D), lambda i: (0, 0)),
            pl.BlockSpec((SDFF, D), lambda i: (0, 0)),
            pl.BlockSpec((D, SDFF), lambda i: (0, 0)),
        ],
        out_specs=pl.BlockSpec((S2 // 8, D), lambda i: (i, 0)),
        out_shape=jax.ShapeDtypeStruct((S2, D), jnp.float32),
    )(xb, shared_gate, shared_up, shared_down)

    out = pl.pallas_call(
        _sweep_body,
        grid=(E,),
        in_specs=[
            pl.BlockSpec((S2, E), lambda e: (0, 0)),
            pl.BlockSpec((S2, D), lambda e: (0, 0)),
            pl.BlockSpec((1, DFF, D), lambda e: (e, 0, 0)),
            pl.BlockSpec((1, DFF, D), lambda e: (e, 0, 0)),
            pl.BlockSpec((1, D, DFF), lambda e: (e, 0, 0)),
            pl.BlockSpec((S2, D), lambda e: (0, 0)),
        ],
        out_specs=pl.BlockSpec((S2, D), lambda e: (0, 0)),
        out_shape=jax.ShapeDtypeStruct((S2, D), jnp.float32),
        scratch_shapes=[pltpu.VMEM((S2, D), jnp.float32)],
    )(w_dense, xb, gate_proj, up_proj, down_proj, shared_out)

    return out.reshape(1, S2, D)


# speedup vs baseline: 22.4052x; 1.0547x over previous
import functools

import jax
import jax.numpy as jnp
from jax import lax
from jax.experimental import pallas as pl
from jax.experimental.pallas import tpu as pltpu
from jax.experimental.pallas import tpu_sc as plsc

S2 = 2048
D = 1024
E = 64
TOPK = 8
NG = 8
TG = 4
DFF = 512
RSF = 2.5
SDFF = 1024

NA = S2 * TOPK
TM = 256
NT = NA // TM + E
NROWS = NT * TM
NW = 32
APW = NA // NW
TPW = S2 // NW
CH = 64

_NEG = -1e30


def _router_body(x_ref, gw_ref, b_ref, idx_ref, w8_ref):
    x = x_ref[...]
    gw = gw_ref[...]
    logits = jax.lax.dot_general(
        x.astype(jnp.bfloat16), gw.astype(jnp.bfloat16),
        (((1,), (1,)), ((), ())),
        preferred_element_type=jnp.float32,
    )
    scores = jax.nn.sigmoid(logits)
    sc = scores + b_ref[...]
    cols = jax.lax.broadcasted_iota(jnp.int32, (S2, E), 1)
    grp = cols // (E // NG)

    gs_full = jnp.zeros_like(sc)
    for g in range(NG):
        ing = grp == g
        vals = jnp.where(ing, sc, _NEG)
        m1 = jnp.max(vals, axis=-1, keepdims=True)
        i1 = jnp.min(jnp.where(vals == m1, cols, 9999), axis=-1, keepdims=True)
        m2 = jnp.max(jnp.where(cols == i1, _NEG, vals), axis=-1, keepdims=True)
        gs_full = gs_full + jnp.where(ing, m1 + m2, 0.0)

    gsr = gs_full
    chosen = jnp.zeros_like(sc, dtype=jnp.bool_)
    for _ in range(TG):
        m = jnp.max(gsr, axis=-1, keepdims=True)
        gidx = jnp.min(jnp.where(gsr == m, grp, 9999), axis=-1, keepdims=True)
        ch = grp == gidx
        chosen = jnp.logical_or(chosen, ch)
        gsr = jnp.where(ch, _NEG, gsr)

    tmp = jnp.where(chosen, sc, 0.0)
    sel = jnp.zeros_like(sc, dtype=jnp.bool_)
    kcols = jax.lax.broadcasted_iota(jnp.int32, (S2, TOPK), 1)
    idx8 = jnp.zeros((S2, TOPK), jnp.int32)
    w8 = jnp.zeros((S2, TOPK), jnp.float32)
    wsum = jnp.zeros((S2, 1), jnp.float32)
    for k in range(TOPK):
        cur = jnp.where(sel, _NEG, tmp)
        m = jnp.max(cur, axis=-1, keepdims=True)
        ik = jnp.min(jnp.where(cur == m, cols, 9999), axis=-1, keepdims=True)
        sel = jnp.logical_or(sel, cols == ik)
        wk = jnp.sum(jnp.where(cols == ik, scores, 0.0), axis=-1,
                     keepdims=True)
        idx8 = idx8 + jnp.where(kcols == k, ik, 0)
        w8 = w8 + jnp.where(kcols == k, wk, 0.0)
        wsum = wsum + wk
    idx_ref[...] = idx8
    w8_ref[...] = w8 * (RSF / (wsum + 1e-20))


def _rank_body(idx_ref, p8_ref, eot_ref, rank_s):
    idx = idx_ref[...]
    cols = jax.lax.broadcasted_iota(jnp.int32, (S2, E), 1)
    oh = jnp.zeros((S2, E), jnp.float32)
    for k in range(TOPK):
        oh = oh + (cols == idx[:, k:k + 1]).astype(jnp.float32)

    r0 = jax.lax.broadcasted_iota(jnp.int32, (128, 128), 0)
    c0 = jax.lax.broadcasted_iota(jnp.int32, (128, 128), 1)
    ls = (r0 > c0).astype(jnp.float32)
    off = jnp.zeros((1, E), jnp.float32)
    for b in range(S2 // 128):
        ohb = oh[b * 128:(b + 1) * 128, :]
        rb = jax.lax.dot_general(ls, ohb, (((1,), (0,)), ((), ())),
                                 preferred_element_type=jnp.float32)
        rank_s[b * 128:(b + 1) * 128, :] = rb + off
        off = off + jnp.sum(ohb, axis=0, keepdims=True)
    counts = off
    tiles = jnp.floor((counts + (TM - 1)) / TM)
    r64 = jax.lax.broadcasted_iota(jnp.int32, (E, E), 0)
    c64 = jax.lax.broadcasted_iota(jnp.int32, (E, E), 1)
    tex = (r64 < c64).astype(jnp.float32)
    base_t = jax.lax.dot_general(tiles, tex, (((1,), (0,)), ((), ())),
                                 preferred_element_type=jnp.float32)
    base_rows = base_t * TM
    p_dense = rank_s[...] + base_rows

    kcols = jax.lax.broadcasted_iota(jnp.int32, (S2, TOPK), 1)
    p8 = jnp.zeros((S2, TOPK), jnp.float32)
    for k in range(TOPK):
        pk = jnp.sum(jnp.where(cols == idx[:, k:k + 1], p_dense, 0.0),
                     axis=-1, keepdims=True)
        p8 = p8 + jnp.where(kcols == k, pk, 0.0)
    p8_ref[...] = p8.astype(jnp.int32)

    jrow = jax.lax.broadcasted_iota(jnp.int32, (1, NT), 1).astype(jnp.float32)
    eotf = jnp.zeros((1, NT), jnp.float32)
    for e in range(E):
        eotf = eotf + (jrow >= base_t[0:1, e:e + 1]).astype(jnp.float32)
    total = jnp.sum(tiles, axis=-1, keepdims=True)
    eot = jnp.where(jrow < total, eotf - 1.0, -1.0)
    eot_ref[...] = eot.astype(jnp.int32)


def _dispatch_body(x_hbm, p_hbm, tok_hbm, wf_hbm, sx_hbm, ws_hbm,
                   tokv, posv, rows, wvv, sem1, sem2, sem3):
    wid = lax.axis_index("s") * 2 + lax.axis_index("c")
    base = wid * APW
    for ch in range(APW // CH):
        off = base + ch * CH
        pltpu.sync_copy(tok_hbm.at[pl.ds(off, CH)], tokv)
        pltpu.sync_copy(p_hbm.at[pl.ds(off, CH)], posv)
        pltpu.sync_copy(wf_hbm.at[pl.ds(off, CH)], wvv)
        pltpu.async_copy(x_hbm.at[tokv], rows, sem1).wait()
        pltpu.async_copy(rows, sx_hbm.at[posv], sem2).wait()
        pltpu.async_copy(wvv, ws_hbm.at[posv], sem3).wait()


def _combine_body(os_hbm, p_hbm, sh_hbm, out_hbm, pidx, rows, shv, acc, sem1):
    wid = lax.axis_index("s") * 2 + lax.axis_index("c")
    tok_base = wid * TPW
    for ch in range(TPW // 8):
        t0 = tok_base + ch * 8
        pltpu.sync_copy(p_hbm.at[pl.ds(t0 * TOPK, 8 * TOPK)], pidx)
        pltpu.async_copy(os_hbm.at[pidx], rows, sem1).wait()
        pltpu.sync_copy(sh_hbm.at[pl.ds(t0, 8)], shv)
        for t in range(8):
            def body(j, _):
                v = shv[t, pl.ds(j * 16, 16)]
                for k in range(TOPK):
                    v = v + rows[TOPK * t + k, pl.ds(j * 16, 16)]
                acc[t, pl.ds(j * 16, 16)] = v
                return 0

            lax.fori_loop(0, D // 16, body, 0)
        pltpu.sync_copy(acc, out_hbm.at[pl.ds(t0, 8)])


def _gemm_body(eot_s, xs_ref, w_ref, g_ref, u_ref, d_ref, o_ref):
    i = pl.program_id(0)

    @pl.when(eot_s[i] >= 0)
    def _():
        xb = xs_ref[...].astype(jnp.bfloat16)
        g = g_ref[0].astype(jnp.bfloat16)
        u = u_ref[0].astype(jnp.bfloat16)
        a = jax.lax.dot_general(xb, g, (((1,), (1,)), ((), ())),
                                preferred_element_type=jnp.float32)
        b = jax.lax.dot_general(xb, u, (((1,), (1,)), ((), ())),
                                preferred_element_type=jnp.float32)
        h = (a * jax.nn.sigmoid(a)) * b
        d = d_ref[0].astype(jnp.bfloat16)
        o = jax.lax.dot_general(h.astype(jnp.bfloat16), d,
                                (((1,), (1,)), ((), ())),
                                preferred_element_type=jnp.float32)
        o_ref[...] = o * w_ref[...]


def _shared_body(x_ref, sg_ref, su_ref, sd_ref, out_ref):
    xb = x_ref[...]
    a = jax.lax.dot_general(xb, sg_ref[...].astype(jnp.bfloat16),
                            (((1,), (1,)), ((), ())),
                            preferred_element_type=jnp.float32)
    b = jax.lax.dot_general(xb, su_ref[...].astype(jnp.bfloat16),
                            (((1,), (1,)), ((), ())),
                            preferred_element_type=jnp.float32)
    h = (a * jax.nn.sigmoid(a)) * b
    out_ref[...] = jax.lax.dot_general(h.astype(jnp.bfloat16),
                                       sd_ref[...].astype(jnp.bfloat16),
                                       (((1,), (1,)), ((), ())),
                                       preferred_element_type=jnp.float32)


def _expert_of(eot_ref, i):
    e = eot_ref[i]
    return jnp.where(e < 0, E - 1, e)


def kernel(hidden_states, gate_weight, e_score_correction_bias, gate_proj,
           up_proj, down_proj, shared_gate, shared_up, shared_down):
    x = hidden_states.reshape(S2, D).astype(jnp.float32)
    bias2d = e_score_correction_bias.reshape(1, E)

    idx8, w8 = pl.pallas_call(
        _router_body,
        out_shape=(jax.ShapeDtypeStruct((S2, TOPK), jnp.int32),
                   jax.ShapeDtypeStruct((S2, TOPK), jnp.float32)),
    )(x, gate_weight, bias2d)

    p8, eot = pl.pallas_call(
        _rank_body,
        out_shape=(jax.ShapeDtypeStruct((S2, TOPK), jnp.int32),
                   jax.ShapeDtypeStruct((1, NT), jnp.int32)),
        scratch_shapes=[pltpu.VMEM((S2, E), jnp.float32)],
    )(idx8)

    p_flat = p8.reshape(NA)
    w_flat = w8.reshape(NA)
    tok_flat = (jnp.arange(NA, dtype=jnp.int32) // TOPK).astype(jnp.int32)

    mesh = plsc.VectorSubcoreMesh(core_axis_name="c", subcore_axis_name="s",
                                  num_cores=2, num_subcores=16)

    sorted_x, w_sorted = pl.kernel(
        _dispatch_body,
        out_type=(jax.ShapeDtypeStruct((NROWS, D), jnp.float32),
                  jax.ShapeDtypeStruct((NROWS,), jnp.float32)),
        mesh=mesh,
        scratch_types=[
            pltpu.VMEM((CH,), jnp.int32),
            pltpu.VMEM((CH,), jnp.int32),
            pltpu.VMEM((CH, D), jnp.float32),
            pltpu.VMEM((CH,), jnp.float32),
            pltpu.SemaphoreType.DMA,
            pltpu.SemaphoreType.DMA,
            pltpu.SemaphoreType.DMA,
        ],
    )(x, p_flat, tok_flat, w_flat)

    xb = x.astype(jnp.bfloat16)
    shared_out = pl.pallas_call(
        _shared_body,
        grid=(8,),
        in_specs=[
            pl.BlockSpec((S2 // 8, D), lambda i: (i, 0)),
            pl.BlockSpec((SDFF, D), lambda i: (0, 0)),
            pl.BlockSpec((SDFF, D), lambda i: (0, 0)),
            pl.BlockSpec((D, SDFF), lambda i: (0, 0)),
        ],
        out_specs=pl.BlockSpec((S2 // 8, D), lambda i: (i, 0)),
        out_shape=jax.ShapeDtypeStruct((S2, D), jnp.float32),
    )(xb, shared_gate, shared_up, shared_down)

    out_sorted = pl.pallas_call(
        _gemm_body,
        grid_spec=pltpu.PrefetchScalarGridSpec(
            num_scalar_prefetch=1,
            grid=(NT,),
            in_specs=[
                pl.BlockSpec((TM, D), lambda i, eot: (i, 0)),
                pl.BlockSpec((TM, 1), lambda i, eot: (i, 0)),
                pl.BlockSpec((1, DFF, D),
                             lambda i, eot: (_expert_of(eot, i), 0, 0)),
                pl.BlockSpec((1, DFF, D),
                             lambda i, eot: (_expert_of(eot, i), 0, 0)),
                pl.BlockSpec((1, D, DFF),
                             lambda i, eot: (_expert_of(eot, i), 0, 0)),
            ],
            out_specs=pl.BlockSpec((TM, D), lambda i, eot: (i, 0)),
        ),
        out_shape=jax.ShapeDtypeStruct((NROWS, D), jnp.float32),
    )(eot.reshape(NT), sorted_x, w_sorted.reshape(NROWS, 1),
      gate_proj, up_proj, down_proj)

    out = pl.kernel(
        _combine_body,
        out_type=jax.ShapeDtypeStruct((S2, D), jnp.float32),
        mesh=mesh,
        scratch_types=[
            pltpu.VMEM((8 * TOPK,), jnp.int32),
            pltpu.VMEM((8 * TOPK, D), jnp.float32),
            pltpu.VMEM((8, D), jnp.float32),
            pltpu.VMEM((8, D), jnp.float32),
            pltpu.SemaphoreType.DMA,
        ],
    )(out_sorted, p_flat, shared_out)

    return out.reshape(1, S2, D)
